# baseline (speedup 1.0000x reference)
import functools
import math

import jax
import jax.numpy as jnp
import numpy as np
from jax import lax
from jax.experimental import pallas as pl
from jax.experimental.pallas import tpu as pltpu
from jax.experimental.pallas import tpu_sc as plsc

N = 50000
E = 800000
B = 16
A = 4
NMAX = 4
RC = 5.0
Q = A * NMAX
HID = 256

_T = 1000
_NBLK = N // _T


def _dense_body(cg0, cg1, cg2, cg3, cg4, wp_ref, w1t_ref, b1_ref, w2t_ref,
                b2_ref, w3_ref, out_ref):
    cgs = [cg0, cg1, cg2, cg3, cg4]
    cm = []
    for m in range(9):
        g, mloc = divmod(m, 2)
        cg = cgs[g][...]
        cm.append(jnp.concatenate(
            [cg[:, a * 8 + mloc * 4: a * 8 + mloc * 4 + NMAX] for a in range(A)],
            axis=1))
    ps_blocks = []
    for l, s, e in ((0, 0, 1), (1, 1, 4), (2, 4, 9)):
        scale = 1.0 / math.sqrt(2 * l + 1)
        acc = None
        for m in range(s, e):
            cl = cm[m]
            term = cl[:, :, None] * cl[:, None, :]
            acc = term if acc is None else acc + term
        ps_blocks.append((acc * scale).reshape(_T, Q * Q))
    ps = jnp.concatenate(ps_blocks, axis=-1)

    psl = jnp.dot(ps, wp_ref[0, :], preferred_element_type=jnp.float32)
    h = jnp.dot(ps, w1t_ref[...], preferred_element_type=jnp.float32) + b1_ref[...]
    h = h * jax.nn.sigmoid(h)
    h = jnp.dot(h, w2t_ref[...], preferred_element_type=jnp.float32) + b2_ref[...]
    h = h * jax.nn.sigmoid(h)
    psnn = jnp.dot(h, w3_ref[0, :], preferred_element_type=jnp.float32)
    out_ref[...] = (psl + psnn)[None, None, :]


def _dense_stage(c5, Wp, W1, b1, W2, b2, W3):
    w1t = W1.T
    w2t = W2.T
    cgs = [c5[g, :_ROWS].reshape(N, A * 8) for g in range(_NGRP)]
    grid = (_NBLK,)
    out = pl.pallas_call(
        _dense_body,
        grid=grid,
        in_specs=[pl.BlockSpec((_T, A * 8), lambda i: (i, 0))] * _NGRP + [
            pl.BlockSpec((1, Q * Q * 3), lambda i: (0, 0)),
            pl.BlockSpec((Q * Q * 3, HID), lambda i: (0, 0)),
            pl.BlockSpec((HID,), lambda i: (0,)),
            pl.BlockSpec((HID, HID), lambda i: (0, 0)),
            pl.BlockSpec((HID,), lambda i: (0,)),
            pl.BlockSpec((1, HID), lambda i: (0, 0)),
        ],
        out_specs=pl.BlockSpec((1, 1, _T), lambda i: (i, 0, 0)),
        out_shape=jax.ShapeDtypeStruct((_NBLK, 1, _T), jnp.float32),
    )(*cgs, Wp, w1t, b1, w2t, b2, W3)
    return out.reshape(N)


_CHUNK = 128
_NCHUNKS = E // _CHUNK
_NW = 32
_TW = 8


def _gather_body(table_hbm, src_hbm, dst_hbm, s_out, d_out, idx_v, rows_v, sem):
    wid = lax.axis_index("s") * 2 + lax.axis_index("c")
    per = _NCHUNKS // _NW
    rem = _NCHUNKS % _NW
    lo = wid * per + jnp.minimum(wid, rem)
    hi = lo + per + (wid < rem).astype(jnp.int32)

    def body(i, carry):
        off = i * _CHUNK
        pltpu.sync_copy(src_hbm.at[pl.ds(off, _CHUNK)], idx_v)
        pltpu.async_copy(table_hbm.at[idx_v], rows_v, sem).wait()
        pltpu.sync_copy(rows_v, s_out.at[pl.ds(off, _CHUNK), :])
        pltpu.sync_copy(dst_hbm.at[pl.ds(off, _CHUNK)], idx_v)
        pltpu.async_copy(table_hbm.at[idx_v], rows_v, sem).wait()
        pltpu.sync_copy(rows_v, d_out.at[pl.ds(off, _CHUNK), :])
        return carry

    lax.fori_loop(lo, hi, body, 0)


def _gather_stage(table, src, dst):
    mesh = plsc.VectorSubcoreMesh(core_axis_name="c", subcore_axis_name="s")
    f = pl.kernel(
        _gather_body,
        mesh=mesh,
        compiler_params=pltpu.CompilerParams(use_tc_tiling_on_sc=False),
        out_type=[
            jax.ShapeDtypeStruct((_EPAD, _TW), jnp.float32),
            jax.ShapeDtypeStruct((_EPAD, _TW), jnp.float32),
        ],
        scratch_types=[
            pltpu.VMEM((_CHUNK,), jnp.int32),
            pltpu.VMEM((_CHUNK, _TW), jnp.float32),
            pltpu.SemaphoreType.DMA,
        ],
    )
    return f(table, src, dst)


_EPAD = 819200
_EB = 4096
_NEB = _EPAD // _EB
_NGRP = 5
_ROWS = N * A
_ROWSP = _ROWS + 16
_RPT = _ROWSP // 16
_SUP = 1280
_NSUP = _EPAD // 16 // _SUP


def _edge_math_body(s_ref, d_ref, dst_ref, out_ref, idx_ref):
    S2 = s_ref[...]
    D2 = d_ref[...]
    li = jax.lax.broadcasted_iota(jnp.int32, (128, 16), 0)
    ki = jax.lax.broadcasted_iota(jnp.int32, (128, 16), 1)
    def pick(M, c):
        P = (li == 8 * ki + c).astype(jnp.float32)
        return jnp.dot(M, P, preferred_element_type=jnp.float32)
    vx = pick(S2, 0) - pick(D2, 0)
    vy = pick(S2, 1) - pick(D2, 1)
    vz = pick(S2, 2) - pick(D2, 2)
    num = pick(S2, 3).astype(jnp.int32)
    r2 = vx * vx + vy * vy + vz * vz
    r = jnp.sqrt(r2 + 1e-12)
    fc = 0.5 * (jnp.cos(jnp.pi * r / RC) + 1.0) * (r < RC).astype(jnp.float32)
    rinv = 1.0 / r
    x = vx * rinv
    y = vy * rinv
    z = vz * rinv
    c0 = 0.28209479177387814
    c1 = 0.4886025119029199
    c2a = 1.0925484305920792
    c2b = 0.31539156525252005
    c2c = 0.5462742152960396
    Ys = [
        jnp.full_like(x, c0),
        c1 * y, c1 * z, c1 * x,
        c2a * x * y, c2a * y * z, c2b * (3.0 * z * z - 1.0),
        c2a * x * z, c2c * (x * x - y * y),
    ]
    mu = np.linspace(0.0, RC, NMAX)
    rads = [jnp.exp(-((r - mu[n]) ** 2)) * fc for n in range(NMAX)]
    groups = []
    for g in range(_NGRP):
        cols = []
        for mloc in range(2):
            m = 2 * g + mloc
            for n in range(NMAX):
                cols.append(rads[n] * Ys[m] if m < 9 else jnp.zeros_like(x))
        groups.append(jnp.stack(cols, axis=-1))
    out_ref[...] = jnp.stack(groups, axis=0)
    i = pl.program_id(0)
    sh = (_EB // 16, 16)
    eloc = (jax.lax.broadcasted_iota(jnp.int32, sh, 0) * 16
            + jax.lax.broadcasted_iota(jnp.int32, sh, 1))
    valid = eloc + i * _EB < E
    idx_ref[0] = jnp.where(valid, dst_ref[0] * A + num, _ROWS)


def _edge_math_stage(S, D, dst):
    out, idx = pl.pallas_call(
        _edge_math_body,
        grid=(_NEB,),
        in_specs=[
            pl.BlockSpec((_EB * _TW // 128, 128), lambda i: (i, 0)),
            pl.BlockSpec((_EB * _TW // 128, 128), lambda i: (i, 0)),
            pl.BlockSpec((1, _EB // 16, 16), lambda i: (i, 0, 0)),
        ],
        out_specs=[
            pl.BlockSpec((_NGRP, _EB // 16, 16, 8), lambda i: (0, i, 0, 0)),
            pl.BlockSpec((1, _EB // 16, 16), lambda i: (i, 0, 0)),
        ],
        out_shape=[
            jax.ShapeDtypeStruct((_NGRP, _EPAD // 16, 16, 8), jnp.float32),
            jax.ShapeDtypeStruct((_NEB, _EB // 16, 16), jnp.int32),
        ],
    )(S.reshape(_EPAD * _TW // 128, 128), D.reshape(_EPAD * _TW // 128, 128),
      jnp.pad(dst, (0, _EPAD - E)).reshape(_NEB, _EB // 16, 16))
    return out.reshape(_NGRP, _EPAD, 8), idx.reshape(_EPAD)


def _scatter_body(contrib_hbm, idx2_hbm, zeros_hbm, out_hbm, acc, ibuf, cbuf, ssem):
    core = lax.axis_index("c")
    sub = lax.axis_index("s")
    for gs in range(3):
        geff = gs + 3 * core

        @pl.when(geff < _NGRP)
        def _():
            pltpu.sync_copy(zeros_hbm, acc.at[pl.ds(sub * _RPT, _RPT), :])
            plsc.subcore_barrier()

            def it(j, carry):
                base = sub * (_EPAD // 16) + j * _SUP
                row = base // 128
                pltpu.sync_copy(idx2_hbm.at[pl.ds(row, 10), :], ibuf)
                pltpu.sync_copy(contrib_hbm.at[geff, pl.ds(base, _SUP), :], cbuf)
                hs = []
                for k in range(10):
                    hs.append(pltpu.async_copy(
                        cbuf.at[pl.ds(k * 128, 128), :],
                        acc.at[ibuf.at[k]], ssem, add=True))
                for h in hs:
                    h.wait()
                return carry

            lax.fori_loop(0, _NSUP, it, 0)
            plsc.subcore_barrier()
            pltpu.sync_copy(acc.at[pl.ds(sub * _RPT, _RPT), :],
                            out_hbm.at[geff, pl.ds(sub * _RPT, _RPT), :])
            plsc.subcore_barrier()


def _scatter_stage(contrib, idx):
    idx2 = idx.reshape(_EPAD // 128, 128)
    zeros = jnp.zeros((_RPT, 8), jnp.float32)
    mesh = plsc.VectorSubcoreMesh(core_axis_name="c", subcore_axis_name="s")
    f = pl.kernel(
        _scatter_body,
        mesh=mesh,
        compiler_params=pltpu.CompilerParams(use_tc_tiling_on_sc=False),
        out_type=jax.ShapeDtypeStruct((_NGRP, _ROWSP, 8), jnp.float32),
        scratch_types=[
            pltpu.VMEM_SHARED((_ROWSP, 8), jnp.float32),
            pltpu.VMEM((10, 128), jnp.int32),
            pltpu.VMEM((_SUP, 8), jnp.float32),
            pltpu.SemaphoreType.DMA,
        ],
    )
    return f(contrib, idx2, zeros)


def _edge_stage(positions, numbers, edge_indices):
    src = edge_indices[0]
    dst = edge_indices[1]
    table = jnp.concatenate(
        [positions, numbers.astype(jnp.float32)[:, None],
         jnp.zeros((N, _TW - 4), jnp.float32)], axis=1)
    S, D = _gather_stage(table, src, dst)
    contrib, idx = _edge_math_stage(S, D, dst)
    return _scatter_stage(contrib, idx)


def kernel(positions, cells, numbers, edge_indices, edge_shifts, ptr,
           Wc, bc, Wp, bp, W1, b1, W2, b2, W3, b3):
    del cells, edge_shifts
    numbers = numbers.astype(jnp.int32)
    edge_indices = edge_indices.astype(jnp.int32)
    one_hot = jax.nn.one_hot(numbers, A, dtype=positions.dtype)
    compositions = one_hot.reshape(B, N // B, A).sum(axis=1)
    energies = compositions @ Wc.T + bc

    c = _edge_stage(positions, numbers, edge_indices)
    eatom = _dense_stage(c, Wp, W1, b1, W2, b2, W3)
    per_struct = eatom.reshape(B, N // B).sum(axis=1)
    extra = jnp.float32(N // B) * (bp[0] + b3[0])
    return energies + (per_struct + extra)[:, None]


# speedup vs baseline: 1.2233x; 1.2233x over previous
import functools
import math

import jax
import jax.numpy as jnp
import numpy as np
from jax import lax
from jax.experimental import pallas as pl
from jax.experimental.pallas import tpu as pltpu
from jax.experimental.pallas import tpu_sc as plsc

N = 50000
E = 800000
B = 16
A = 4
NMAX = 4
RC = 5.0
Q = A * NMAX
HID = 256

_T = 1000
_NBLK = N // _T


def _dense_body(cg0, cg1, cg2, cg3, cg4, wp_ref, w1t_ref, b1_ref, w2t_ref,
                b2_ref, w3_ref, out_ref):
    cgs = [cg0, cg1, cg2, cg3, cg4]
    cm = []
    for m in range(9):
        g, mloc = divmod(m, 2)
        cg = cgs[g][...]
        cm.append(jnp.concatenate(
            [cg[:, a * 8 + mloc * 4: a * 8 + mloc * 4 + NMAX] for a in range(A)],
            axis=1))
    ps_blocks = []
    for l, s, e in ((0, 0, 1), (1, 1, 4), (2, 4, 9)):
        scale = 1.0 / math.sqrt(2 * l + 1)
        acc = None
        for m in range(s, e):
            cl = cm[m]
            term = cl[:, :, None] * cl[:, None, :]
            acc = term if acc is None else acc + term
        ps_blocks.append((acc * scale).reshape(_T, Q * Q))
    ps = jnp.concatenate(ps_blocks, axis=-1)

    psl = jnp.dot(ps, wp_ref[0, :], preferred_element_type=jnp.float32)
    h = jnp.dot(ps, w1t_ref[...], preferred_element_type=jnp.float32) + b1_ref[...]
    h = h * jax.nn.sigmoid(h)
    h = jnp.dot(h, w2t_ref[...], preferred_element_type=jnp.float32) + b2_ref[...]
    h = h * jax.nn.sigmoid(h)
    psnn = jnp.dot(h, w3_ref[0, :], preferred_element_type=jnp.float32)
    out_ref[...] = (psl + psnn)[None, None, :]


def _dense_stage(c5, Wp, W1, b1, W2, b2, W3):
    w1t = W1.T
    w2t = W2.T
    cgs = [c5[g, :_ROWS].reshape(N, A * 8) for g in range(_NGRP)]
    grid = (_NBLK,)
    out = pl.pallas_call(
        _dense_body,
        grid=grid,
        in_specs=[pl.BlockSpec((_T, A * 8), lambda i: (i, 0))] * _NGRP + [
            pl.BlockSpec((1, Q * Q * 3), lambda i: (0, 0)),
            pl.BlockSpec((Q * Q * 3, HID), lambda i: (0, 0)),
            pl.BlockSpec((HID,), lambda i: (0,)),
            pl.BlockSpec((HID, HID), lambda i: (0, 0)),
            pl.BlockSpec((HID,), lambda i: (0,)),
            pl.BlockSpec((1, HID), lambda i: (0, 0)),
        ],
        out_specs=pl.BlockSpec((1, 1, _T), lambda i: (i, 0, 0)),
        out_shape=jax.ShapeDtypeStruct((_NBLK, 1, _T), jnp.float32),
    )(*cgs, Wp, w1t, b1, w2t, b2, W3)
    return out.reshape(N)


_CHUNK = 128
_NCHUNKS = E // _CHUNK
_NW = 32
_TW = 16


def _gather_body(table_hbm, src_hbm, dst_hbm, s_out, d_out, idx_v, rows_v, sem):
    wid = lax.axis_index("s") * 2 + lax.axis_index("c")
    per = _NCHUNKS // _NW
    rem = _NCHUNKS % _NW
    lo = wid * per + jnp.minimum(wid, rem)
    hi = lo + per + (wid < rem).astype(jnp.int32)

    def body(i, carry):
        off = i * _CHUNK
        pltpu.sync_copy(src_hbm.at[pl.ds(off, _CHUNK)], idx_v)
        pltpu.async_copy(table_hbm.at[idx_v], rows_v, sem).wait()
        pltpu.sync_copy(rows_v, s_out.at[pl.ds(off, _CHUNK), :])
        pltpu.sync_copy(dst_hbm.at[pl.ds(off, _CHUNK)], idx_v)
        pltpu.async_copy(table_hbm.at[idx_v], rows_v, sem).wait()
        pltpu.sync_copy(rows_v, d_out.at[pl.ds(off, _CHUNK), :])
        return carry

    lax.fori_loop(lo, hi, body, 0)


def _gather_stage(table, src, dst):
    mesh = plsc.VectorSubcoreMesh(core_axis_name="c", subcore_axis_name="s")
    f = pl.kernel(
        _gather_body,
        mesh=mesh,
        compiler_params=pltpu.CompilerParams(use_tc_tiling_on_sc=False),
        out_type=[
            jax.ShapeDtypeStruct((E, _TW), jnp.float32),
            jax.ShapeDtypeStruct((E, _TW), jnp.float32),
        ],
        scratch_types=[
            pltpu.VMEM((_CHUNK,), jnp.int32),
            pltpu.VMEM((_CHUNK, _TW), jnp.float32),
            pltpu.SemaphoreType.DMA,
        ],
    )
    return f(table, src, dst)


_EPAD = 819200
_EB = 16384
_NEB = _EPAD // _EB
_NGRP = 5
_ROWS = N * A
_ROWSP = _ROWS + 16
_RPT = _ROWSP // 16
_SUP = 1280
_NSUP = _EPAD // 16 // _SUP


def _edge_math_body(s_ref, d_ref, dst_ref, out_ref, idx_ref):
    vx = s_ref[0] - d_ref[0]
    vy = s_ref[1] - d_ref[1]
    vz = s_ref[2] - d_ref[2]
    num = s_ref[3].astype(jnp.int32)
    r2 = vx * vx + vy * vy + vz * vz
    r = jnp.sqrt(r2 + 1e-12)
    fc = 0.5 * (jnp.cos(jnp.pi * r / RC) + 1.0) * (r < RC).astype(jnp.float32)
    rinv = 1.0 / r
    x = vx * rinv
    y = vy * rinv
    z = vz * rinv
    c0 = 0.28209479177387814
    c1 = 0.4886025119029199
    c2a = 1.0925484305920792
    c2b = 0.31539156525252005
    c2c = 0.5462742152960396
    Ys = [
        jnp.full_like(x, c0),
        c1 * y, c1 * z, c1 * x,
        c2a * x * y, c2a * y * z, c2b * (3.0 * z * z - 1.0),
        c2a * x * z, c2c * (x * x - y * y),
    ]
    mu = np.linspace(0.0, RC, NMAX)
    rads = [jnp.exp(-((r - mu[n]) ** 2)) * fc for n in range(NMAX)]
    groups = []
    for g in range(_NGRP):
        cols = []
        for mloc in range(2):
            m = 2 * g + mloc
            for n in range(NMAX):
                cols.append(rads[n] * Ys[m] if m < 9 else jnp.zeros_like(x))
        groups.append(jnp.stack(cols, axis=0))
    out_ref[...] = jnp.stack(groups, axis=0)
    i = pl.program_id(0)
    rowid = (jax.lax.broadcasted_iota(jnp.int32, (_EB // 128, 128), 0)
             + i * (_EB // 128))
    valid = rowid < (E // 128)
    idx_ref[0] = jnp.where(valid, dst_ref[0] * A + num, _ROWS)


def _edge_math_stage(S, D, dst):
    out, idx = pl.pallas_call(
        _edge_math_body,
        grid=(_NEB,),
        in_specs=[
            pl.BlockSpec((_TW, _EB // 128, 128), lambda i: (0, i, 0)),
            pl.BlockSpec((_TW, _EB // 128, 128), lambda i: (0, i, 0)),
            pl.BlockSpec((1, _EB // 128, 128), lambda i: (i, 0, 0)),
        ],
        out_specs=[
            pl.BlockSpec((_NGRP, 8, _EB // 128, 128), lambda i: (0, 0, i, 0)),
            pl.BlockSpec((1, _EB // 128, 128), lambda i: (i, 0, 0)),
        ],
        out_shape=[
            jax.ShapeDtypeStruct((_NGRP, 8, _EPAD // 128, 128), jnp.float32),
            jax.ShapeDtypeStruct((_NEB, _EB // 128, 128), jnp.int32),
        ],
    )(jnp.pad(S.T, ((0, 0), (0, _EPAD - E))).reshape(_TW, _EPAD // 128, 128),
      jnp.pad(D.T, ((0, 0), (0, _EPAD - E))).reshape(_TW, _EPAD // 128, 128),
      jnp.pad(dst, (0, _EPAD - E)).reshape(_NEB, _EB // 128, 128))
    return (jnp.transpose(out.reshape(_NGRP, 8, _EPAD), (0, 2, 1)),
            idx.reshape(_EPAD))


def _scatter_body(contrib_hbm, idx2_hbm, zeros_hbm, out_hbm, acc, ibuf, cbuf, ssem):
    core = lax.axis_index("c")
    sub = lax.axis_index("s")
    for gs in range(3):
        geff = gs + 3 * core

        @pl.when(geff < _NGRP)
        def _():
            pltpu.sync_copy(zeros_hbm, acc.at[pl.ds(sub * _RPT, _RPT), :])
            plsc.subcore_barrier()

            def it(j, carry):
                base = sub * (_EPAD // 16) + j * _SUP
                row = base // 128
                pltpu.sync_copy(idx2_hbm.at[pl.ds(row, 10), :], ibuf)
                pltpu.sync_copy(contrib_hbm.at[geff, pl.ds(base, _SUP), :], cbuf)
                hs = []
                for k in range(10):
                    hs.append(pltpu.async_copy(
                        cbuf.at[pl.ds(k * 128, 128), :],
                        acc.at[ibuf.at[k]], ssem, add=True))
                for h in hs:
                    h.wait()
                return carry

            lax.fori_loop(0, _NSUP, it, 0)
            plsc.subcore_barrier()
            pltpu.sync_copy(acc.at[pl.ds(sub * _RPT, _RPT), :],
                            out_hbm.at[geff, pl.ds(sub * _RPT, _RPT), :])
            plsc.subcore_barrier()


def _scatter_stage(contrib, idx):
    idx2 = idx.reshape(_EPAD // 128, 128)
    zeros = jnp.zeros((_RPT, 8), jnp.float32)
    mesh = plsc.VectorSubcoreMesh(core_axis_name="c", subcore_axis_name="s")
    f = pl.kernel(
        _scatter_body,
        mesh=mesh,
        compiler_params=pltpu.CompilerParams(use_tc_tiling_on_sc=False),
        out_type=jax.ShapeDtypeStruct((_NGRP, _ROWSP, 8), jnp.float32),
        scratch_types=[
            pltpu.VMEM_SHARED((_ROWSP, 8), jnp.float32),
            pltpu.VMEM((10, 128), jnp.int32),
            pltpu.VMEM((_SUP, 8), jnp.float32),
            pltpu.SemaphoreType.DMA,
        ],
    )
    return f(contrib, idx2, zeros)


def _edge_stage(positions, numbers, edge_indices):
    src = edge_indices[0]
    dst = edge_indices[1]
    table = jnp.concatenate(
        [positions, numbers.astype(jnp.float32)[:, None],
         jnp.zeros((N, _TW - 4), jnp.float32)], axis=1)
    S, D = _gather_stage(table, src, dst)
    contrib, idx = _edge_math_stage(S, D, dst)
    return _scatter_stage(contrib, idx)


def kernel(positions, cells, numbers, edge_indices, edge_shifts, ptr,
           Wc, bc, Wp, bp, W1, b1, W2, b2, W3, b3):
    del cells, edge_shifts
    numbers = numbers.astype(jnp.int32)
    edge_indices = edge_indices.astype(jnp.int32)
    one_hot = jax.nn.one_hot(numbers, A, dtype=positions.dtype)
    compositions = one_hot.reshape(B, N // B, A).sum(axis=1)
    energies = compositions @ Wc.T + bc

    c = _edge_stage(positions, numbers, edge_indices)
    eatom = _dense_stage(c, Wp, W1, b1, W2, b2, W3)
    per_struct = eatom.reshape(B, N // B).sum(axis=1)
    extra = jnp.float32(N // B) * (bp[0] + b3[0])
    return energies + (per_struct + extra)[:, None]


# speedup vs baseline: 1.2609x; 1.0307x over previous
import functools
import math

import jax
import jax.numpy as jnp
import numpy as np
from jax import lax
from jax.experimental import pallas as pl
from jax.experimental.pallas import tpu as pltpu
from jax.experimental.pallas import tpu_sc as plsc

N = 50000
E = 800000
B = 16
A = 4
NMAX = 4
RC = 5.0
Q = A * NMAX
HID = 256

_T = 1000
_NBLK = N // _T


def _dense_body(cg0, cg1, cg2, cg3, cg4, wp_ref, w1t_ref, b1_ref, w2t_ref,
                b2_ref, w3_ref, out_ref):
    cgs = [cg0, cg1, cg2, cg3, cg4]
    cm = []
    for m in range(9):
        g, mloc = divmod(m, 2)
        cg = cgs[g][...]
        cm.append(jnp.concatenate(
            [cg[:, a * 8 + mloc * 4: a * 8 + mloc * 4 + NMAX] for a in range(A)],
            axis=1))
    ps_blocks = []
    for l, s, e in ((0, 0, 1), (1, 1, 4), (2, 4, 9)):
        scale = 1.0 / math.sqrt(2 * l + 1)
        acc = None
        for m in range(s, e):
            cl = cm[m]
            term = cl[:, :, None] * cl[:, None, :]
            acc = term if acc is None else acc + term
        ps_blocks.append((acc * scale).reshape(_T, Q * Q))
    ps = jnp.concatenate(ps_blocks, axis=-1)

    psl = jnp.dot(ps, wp_ref[0, :], preferred_element_type=jnp.float32)
    h = jnp.dot(ps, w1t_ref[...], preferred_element_type=jnp.float32) + b1_ref[...]
    h = h * jax.nn.sigmoid(h)
    h = jnp.dot(h, w2t_ref[...], preferred_element_type=jnp.float32) + b2_ref[...]
    h = h * jax.nn.sigmoid(h)
    psnn = jnp.dot(h, w3_ref[0, :], preferred_element_type=jnp.float32)
    out_ref[...] = (psl + psnn)[None, None, :]


def _dense_stage(c5, Wp, W1, b1, W2, b2, W3):
    w1t = W1.T
    w2t = W2.T
    cgs = [c5[g, :_ROWS].reshape(N, A * 8) for g in range(_NGRP)]
    grid = (_NBLK,)
    out = pl.pallas_call(
        _dense_body,
        grid=grid,
        in_specs=[pl.BlockSpec((_T, A * 8), lambda i: (i, 0))] * _NGRP + [
            pl.BlockSpec((1, Q * Q * 3), lambda i: (0, 0)),
            pl.BlockSpec((Q * Q * 3, HID), lambda i: (0, 0)),
            pl.BlockSpec((HID,), lambda i: (0,)),
            pl.BlockSpec((HID, HID), lambda i: (0, 0)),
            pl.BlockSpec((HID,), lambda i: (0,)),
            pl.BlockSpec((1, HID), lambda i: (0, 0)),
        ],
        out_specs=pl.BlockSpec((1, 1, _T), lambda i: (i, 0, 0)),
        out_shape=jax.ShapeDtypeStruct((_NBLK, 1, _T), jnp.float32),
    )(*cgs, Wp, w1t, b1, w2t, b2, W3)
    return out.reshape(N)


_CHUNK = 128
_NCHUNKS = E // _CHUNK
_NW = 32
_TW = 16


def _gather_body(table_hbm, src_hbm, dst_hbm, s_out, d_out,
                 idx_s, idx_d, rows_s, rows_d, sem_a, sem_b, sem_c):
    wid = lax.axis_index("s") * 2 + lax.axis_index("c")
    per = _NCHUNKS // _NW
    rem = _NCHUNKS % _NW
    lo = wid * per + jnp.minimum(wid, rem)
    hi = lo + per + (wid < rem).astype(jnp.int32)

    def body(i, carry):
        off = i * _CHUNK
        c1 = pltpu.async_copy(src_hbm.at[pl.ds(off, _CHUNK)], idx_s, sem_a)
        c2 = pltpu.async_copy(dst_hbm.at[pl.ds(off, _CHUNK)], idx_d, sem_a)
        c1.wait()
        g1 = pltpu.async_copy(table_hbm.at[idx_s], rows_s, sem_b)
        c2.wait()
        g2 = pltpu.async_copy(table_hbm.at[idx_d], rows_d, sem_b)
        g1.wait()
        w1 = pltpu.async_copy(rows_s, s_out.at[pl.ds(off, _CHUNK), :], sem_c)
        g2.wait()
        w2 = pltpu.async_copy(rows_d, d_out.at[pl.ds(off, _CHUNK), :], sem_c)
        w1.wait()
        w2.wait()
        return carry

    lax.fori_loop(lo, hi, body, 0)


def _gather_stage(table, src, dst):
    mesh = plsc.VectorSubcoreMesh(core_axis_name="c", subcore_axis_name="s")
    f = pl.kernel(
        _gather_body,
        mesh=mesh,
        compiler_params=pltpu.CompilerParams(use_tc_tiling_on_sc=False),
        out_type=[
            jax.ShapeDtypeStruct((E, _TW), jnp.float32),
            jax.ShapeDtypeStruct((E, _TW), jnp.float32),
        ],
        scratch_types=[
            pltpu.VMEM((_CHUNK,), jnp.int32),
            pltpu.VMEM((_CHUNK,), jnp.int32),
            pltpu.VMEM((_CHUNK, _TW), jnp.float32),
            pltpu.VMEM((_CHUNK, _TW), jnp.float32),
            pltpu.SemaphoreType.DMA,
            pltpu.SemaphoreType.DMA,
            pltpu.SemaphoreType.DMA,
        ],
    )
    return f(table, src, dst)


_EPAD = 819200
_EB = 16384
_NEB = _EPAD // _EB
_NGRP = 5
_ROWS = N * A
_ROWSP = _ROWS + 16
_RPT = _ROWSP // 16
_SUP = 1280
_NSUP = _EPAD // 16 // _SUP


def _edge_math_body(s_ref, d_ref, dst_ref, out_ref, idx_ref):
    vx = s_ref[0] - d_ref[0]
    vy = s_ref[1] - d_ref[1]
    vz = s_ref[2] - d_ref[2]
    num = s_ref[3].astype(jnp.int32)
    r2 = vx * vx + vy * vy + vz * vz
    r = jnp.sqrt(r2 + 1e-12)
    fc = 0.5 * (jnp.cos(jnp.pi * r / RC) + 1.0) * (r < RC).astype(jnp.float32)
    rinv = 1.0 / r
    x = vx * rinv
    y = vy * rinv
    z = vz * rinv
    c0 = 0.28209479177387814
    c1 = 0.4886025119029199
    c2a = 1.0925484305920792
    c2b = 0.31539156525252005
    c2c = 0.5462742152960396
    Ys = [
        jnp.full_like(x, c0),
        c1 * y, c1 * z, c1 * x,
        c2a * x * y, c2a * y * z, c2b * (3.0 * z * z - 1.0),
        c2a * x * z, c2c * (x * x - y * y),
    ]
    mu = np.linspace(0.0, RC, NMAX)
    rads = [jnp.exp(-((r - mu[n]) ** 2)) * fc for n in range(NMAX)]
    groups = []
    for g in range(_NGRP):
        cols = []
        for mloc in range(2):
            m = 2 * g + mloc
            for n in range(NMAX):
                cols.append(rads[n] * Ys[m] if m < 9 else jnp.zeros_like(x))
        groups.append(jnp.stack(cols, axis=0))
    out_ref[...] = jnp.stack(groups, axis=0)
    i = pl.program_id(0)
    rowid = (jax.lax.broadcasted_iota(jnp.int32, (_EB // 128, 128), 0)
             + i * (_EB // 128))
    valid = rowid < (E // 128)
    idx_ref[0] = jnp.where(valid, dst_ref[0] * A + num, _ROWS)


def _edge_math_stage(S, D, dst):
    out, idx = pl.pallas_call(
        _edge_math_body,
        grid=(_NEB,),
        in_specs=[
            pl.BlockSpec((_TW, _EB // 128, 128), lambda i: (0, i, 0)),
            pl.BlockSpec((_TW, _EB // 128, 128), lambda i: (0, i, 0)),
            pl.BlockSpec((1, _EB // 128, 128), lambda i: (i, 0, 0)),
        ],
        out_specs=[
            pl.BlockSpec((_NGRP, 8, _EB // 128, 128), lambda i: (0, 0, i, 0)),
            pl.BlockSpec((1, _EB // 128, 128), lambda i: (i, 0, 0)),
        ],
        out_shape=[
            jax.ShapeDtypeStruct((_NGRP, 8, _EPAD // 128, 128), jnp.float32),
            jax.ShapeDtypeStruct((_NEB, _EB // 128, 128), jnp.int32),
        ],
    )(jnp.pad(S.T, ((0, 0), (0, _EPAD - E))).reshape(_TW, _EPAD // 128, 128),
      jnp.pad(D.T, ((0, 0), (0, _EPAD - E))).reshape(_TW, _EPAD // 128, 128),
      jnp.pad(dst, (0, _EPAD - E)).reshape(_NEB, _EB // 128, 128))
    return (jnp.transpose(out.reshape(_NGRP, 8, _EPAD), (0, 2, 1)),
            idx.reshape(_EPAD))


def _scatter_body(contrib_hbm, idx2_hbm, zeros_hbm, out_hbm, acc, ibuf, cbuf, ssem):
    core = lax.axis_index("c")
    sub = lax.axis_index("s")
    for gs in range(3):
        geff = gs + 3 * core

        @pl.when(geff < _NGRP)
        def _():
            pltpu.sync_copy(zeros_hbm, acc.at[pl.ds(sub * _RPT, _RPT), :])
            plsc.subcore_barrier()

            def it(j, carry):
                base = sub * (_EPAD // 16) + j * _SUP
                row = base // 128
                pltpu.sync_copy(idx2_hbm.at[pl.ds(row, 10), :], ibuf)
                pltpu.sync_copy(contrib_hbm.at[geff, pl.ds(base, _SUP), :], cbuf)
                hs = []
                for k in range(10):
                    hs.append(pltpu.async_copy(
                        cbuf.at[pl.ds(k * 128, 128), :],
                        acc.at[ibuf.at[k]], ssem, add=True))
                for h in hs:
                    h.wait()
                return carry

            lax.fori_loop(0, _NSUP, it, 0)
            plsc.subcore_barrier()
            pltpu.sync_copy(acc.at[pl.ds(sub * _RPT, _RPT), :],
                            out_hbm.at[geff, pl.ds(sub * _RPT, _RPT), :])
            plsc.subcore_barrier()


def _scatter_stage(contrib, idx):
    idx2 = idx.reshape(_EPAD // 128, 128)
    zeros = jnp.zeros((_RPT, 8), jnp.float32)
    mesh = plsc.VectorSubcoreMesh(core_axis_name="c", subcore_axis_name="s")
    f = pl.kernel(
        _scatter_body,
        mesh=mesh,
        compiler_params=pltpu.CompilerParams(use_tc_tiling_on_sc=False),
        out_type=jax.ShapeDtypeStruct((_NGRP, _ROWSP, 8), jnp.float32),
        scratch_types=[
            pltpu.VMEM_SHARED((_ROWSP, 8), jnp.float32),
            pltpu.VMEM((10, 128), jnp.int32),
            pltpu.VMEM((_SUP, 8), jnp.float32),
            pltpu.SemaphoreType.DMA,
        ],
    )
    return f(contrib, idx2, zeros)


def _edge_stage(positions, numbers, edge_indices):
    src = edge_indices[0]
    dst = edge_indices[1]
    table = jnp.concatenate(
        [positions, numbers.astype(jnp.float32)[:, None],
         jnp.zeros((N, _TW - 4), jnp.float32)], axis=1)
    S, D = _gather_stage(table, src, dst)
    contrib, idx = _edge_math_stage(S, D, dst)
    return _scatter_stage(contrib, idx)


def kernel(positions, cells, numbers, edge_indices, edge_shifts, ptr,
           Wc, bc, Wp, bp, W1, b1, W2, b2, W3, b3):
    del cells, edge_shifts
    numbers = numbers.astype(jnp.int32)
    edge_indices = edge_indices.astype(jnp.int32)
    one_hot = jax.nn.one_hot(numbers, A, dtype=positions.dtype)
    compositions = one_hot.reshape(B, N // B, A).sum(axis=1)
    energies = compositions @ Wc.T + bc

    c = _edge_stage(positions, numbers, edge_indices)
    eatom = _dense_stage(c, Wp, W1, b1, W2, b2, W3)
    per_struct = eatom.reshape(B, N // B).sum(axis=1)
    extra = jnp.float32(N // B) * (bp[0] + b3[0])
    return energies + (per_struct + extra)[:, None]


# speedup vs baseline: 1.2723x; 1.0091x over previous
import functools
import math

import jax
import jax.numpy as jnp
import numpy as np
from jax import lax
from jax.experimental import pallas as pl
from jax.experimental.pallas import tpu as pltpu
from jax.experimental.pallas import tpu_sc as plsc

N = 50000
E = 800000
B = 16
A = 4
NMAX = 4
RC = 5.0
Q = A * NMAX
HID = 256

_T = 1000
_NBLK = N // _T


def _dense_body(cg0, cg1, cg2, cg3, cg4, wp_ref, w1t_ref, b1_ref, w2t_ref,
                b2_ref, w3_ref, out_ref):
    cgs = [cg0, cg1, cg2, cg3, cg4]
    cm = []
    for m in range(9):
        g, mloc = divmod(m, 2)
        cg = cgs[g][...]
        cm.append(jnp.concatenate(
            [cg[:, a * 8 + mloc * 4: a * 8 + mloc * 4 + NMAX] for a in range(A)],
            axis=1))
    ps_blocks = []
    for l, s, e in ((0, 0, 1), (1, 1, 4), (2, 4, 9)):
        scale = 1.0 / math.sqrt(2 * l + 1)
        acc = None
        for m in range(s, e):
            cl = cm[m]
            term = cl[:, :, None] * cl[:, None, :]
            acc = term if acc is None else acc + term
        ps_blocks.append((acc * scale).reshape(_T, Q * Q))
    ps = jnp.concatenate(ps_blocks, axis=-1)

    psl = jnp.dot(ps, wp_ref[0, :], preferred_element_type=jnp.float32)
    h = jnp.dot(ps, w1t_ref[...], preferred_element_type=jnp.float32) + b1_ref[...]
    h = h * jax.nn.sigmoid(h)
    h = jnp.dot(h, w2t_ref[...], preferred_element_type=jnp.float32) + b2_ref[...]
    h = h * jax.nn.sigmoid(h)
    psnn = jnp.dot(h, w3_ref[0, :], preferred_element_type=jnp.float32)
    out_ref[...] = (psl + psnn)[None, None, :]


def _dense_stage(c5, Wp, W1, b1, W2, b2, W3):
    w1t = W1.T
    w2t = W2.T
    cgs = [c5[g, :_ROWS].reshape(N, A * 8) for g in range(_NGRP)]
    grid = (_NBLK,)
    out = pl.pallas_call(
        _dense_body,
        grid=grid,
        in_specs=[pl.BlockSpec((_T, A * 8), lambda i: (i, 0))] * _NGRP + [
            pl.BlockSpec((1, Q * Q * 3), lambda i: (0, 0)),
            pl.BlockSpec((Q * Q * 3, HID), lambda i: (0, 0)),
            pl.BlockSpec((HID,), lambda i: (0,)),
            pl.BlockSpec((HID, HID), lambda i: (0, 0)),
            pl.BlockSpec((HID,), lambda i: (0,)),
            pl.BlockSpec((1, HID), lambda i: (0, 0)),
        ],
        out_specs=pl.BlockSpec((1, 1, _T), lambda i: (i, 0, 0)),
        out_shape=jax.ShapeDtypeStruct((_NBLK, 1, _T), jnp.float32),
    )(*cgs, Wp, w1t, b1, w2t, b2, W3)
    return out.reshape(N)


_CHUNK = 128
_NCHUNKS = E // _CHUNK
_NW = 32
_TW = 16


def _gather_body(table_hbm, src_hbm, dst_hbm, s_out, d_out,
                 idx_s, idx_d, rows_s, rows_d, sem_a, sem_b, sem_c):
    wid = lax.axis_index("s") * 2 + lax.axis_index("c")
    per = _NCHUNKS // _NW
    rem = _NCHUNKS % _NW
    lo = wid * per + jnp.minimum(wid, rem)
    hi = lo + per + (wid < rem).astype(jnp.int32)

    def body(i, carry):
        off = i * _CHUNK
        c1 = pltpu.async_copy(src_hbm.at[pl.ds(off, _CHUNK)], idx_s, sem_a)
        c2 = pltpu.async_copy(dst_hbm.at[pl.ds(off, _CHUNK)], idx_d, sem_a)
        c1.wait()
        g1 = pltpu.async_copy(table_hbm.at[idx_s], rows_s, sem_b)
        c2.wait()
        g2 = pltpu.async_copy(table_hbm.at[idx_d], rows_d, sem_b)
        g1.wait()
        w1 = pltpu.async_copy(rows_s, s_out.at[pl.ds(off, _CHUNK), :], sem_c)
        g2.wait()
        w2 = pltpu.async_copy(rows_d, d_out.at[pl.ds(off, _CHUNK), :], sem_c)
        w1.wait()
        w2.wait()
        return carry

    lax.fori_loop(lo, hi, body, 0)


def _gather_stage(table, src, dst):
    mesh = plsc.VectorSubcoreMesh(core_axis_name="c", subcore_axis_name="s")
    f = pl.kernel(
        _gather_body,
        mesh=mesh,
        compiler_params=pltpu.CompilerParams(use_tc_tiling_on_sc=False),
        out_type=[
            jax.ShapeDtypeStruct((E, _TW), jnp.float32),
            jax.ShapeDtypeStruct((E, _TW), jnp.float32),
        ],
        scratch_types=[
            pltpu.VMEM((_CHUNK,), jnp.int32),
            pltpu.VMEM((_CHUNK,), jnp.int32),
            pltpu.VMEM((_CHUNK, _TW), jnp.float32),
            pltpu.VMEM((_CHUNK, _TW), jnp.float32),
            pltpu.SemaphoreType.DMA,
            pltpu.SemaphoreType.DMA,
            pltpu.SemaphoreType.DMA,
        ],
    )
    return f(table, src, dst)


_EPAD = 819200
_EB = 16384
_NEB = _EPAD // _EB
_NGRP = 5
_ROWS = N * A
_ROWSP = _ROWS + 16
_RPT = _ROWSP // 16
_SUP = 1280
_NSUP = _EPAD // 16 // _SUP


def _edge_math_body(s_ref, d_ref, dst_ref, out_ref, idx_ref):
    vx = s_ref[0] - d_ref[0]
    vy = s_ref[1] - d_ref[1]
    vz = s_ref[2] - d_ref[2]
    num = s_ref[3].astype(jnp.int32)
    r2 = vx * vx + vy * vy + vz * vz
    r = jnp.sqrt(r2 + 1e-12)
    fc = 0.5 * (jnp.cos(jnp.pi * r / RC) + 1.0) * (r < RC).astype(jnp.float32)
    rinv = 1.0 / r
    x = vx * rinv
    y = vy * rinv
    z = vz * rinv
    c0 = 0.28209479177387814
    c1 = 0.4886025119029199
    c2a = 1.0925484305920792
    c2b = 0.31539156525252005
    c2c = 0.5462742152960396
    Ys = [
        jnp.full_like(x, c0),
        c1 * y, c1 * z, c1 * x,
        c2a * x * y, c2a * y * z, c2b * (3.0 * z * z - 1.0),
        c2a * x * z, c2c * (x * x - y * y),
    ]
    mu = np.linspace(0.0, RC, NMAX)
    rads = [jnp.exp(-((r - mu[n]) ** 2)) * fc for n in range(NMAX)]
    groups = []
    for g in range(_NGRP):
        cols = []
        for mloc in range(2):
            m = 2 * g + mloc
            for n in range(NMAX):
                cols.append(rads[n] * Ys[m] if m < 9 else jnp.zeros_like(x))
        groups.append(jnp.stack(cols, axis=0))
    out_ref[...] = jnp.stack(groups, axis=0)
    i = pl.program_id(0)
    rowid = (jax.lax.broadcasted_iota(jnp.int32, (_EB // 128, 128), 0)
             + i * (_EB // 128))
    valid = rowid < (E // 128)
    idx_ref[0] = jnp.where(valid, dst_ref[0] * A + num, _ROWS)


def _edge_math_stage(S, D, dst):
    out, idx = pl.pallas_call(
        _edge_math_body,
        grid=(_NEB,),
        in_specs=[
            pl.BlockSpec((_TW, _EB // 128, 128), lambda i: (0, i, 0)),
            pl.BlockSpec((_TW, _EB // 128, 128), lambda i: (0, i, 0)),
            pl.BlockSpec((1, _EB // 128, 128), lambda i: (i, 0, 0)),
        ],
        out_specs=[
            pl.BlockSpec((_NGRP, 8, _EB // 128, 128), lambda i: (0, 0, i, 0)),
            pl.BlockSpec((1, _EB // 128, 128), lambda i: (i, 0, 0)),
        ],
        out_shape=[
            jax.ShapeDtypeStruct((_NGRP, 8, _EPAD // 128, 128), jnp.float32),
            jax.ShapeDtypeStruct((_NEB, _EB // 128, 128), jnp.int32),
        ],
    )(jnp.pad(S.T, ((0, 0), (0, _EPAD - E))).reshape(_TW, _EPAD // 128, 128),
      jnp.pad(D.T, ((0, 0), (0, _EPAD - E))).reshape(_TW, _EPAD // 128, 128),
      jnp.pad(dst, (0, _EPAD - E)).reshape(_NEB, _EB // 128, 128))
    return (jnp.transpose(out.reshape(_NGRP, 8, _EPAD), (0, 2, 1)),
            idx.reshape(_EPAD))


def _scatter_body(contrib_hbm, idx2_hbm, zeros_hbm, out_hbm, acc, ibuf, cbuf,
                  ssem, sem_i, sem_c):
    core = lax.axis_index("c")
    sub = lax.axis_index("s")
    for gs in range(3):
        geff = gs + 3 * core

        @pl.when(geff < _NGRP)
        def _():
            pltpu.sync_copy(zeros_hbm, acc.at[pl.ds(sub * _RPT, _RPT), :])
            plsc.subcore_barrier()

            def it(j, carry):
                base = sub * (_EPAD // 16) + j * _SUP
                row = base // 128
                c1 = pltpu.async_copy(idx2_hbm.at[pl.ds(row, 10), :], ibuf, sem_i)
                c2 = pltpu.async_copy(
                    contrib_hbm.at[geff, pl.ds(base, _SUP), :], cbuf, sem_c)
                c1.wait()
                c2.wait()
                hs = []
                for k in range(10):
                    hs.append(pltpu.async_copy(
                        cbuf.at[pl.ds(k * 128, 128), :],
                        acc.at[ibuf.at[k]], ssem, add=True))
                for h in hs:
                    h.wait()
                return carry

            lax.fori_loop(0, _NSUP, it, 0)
            plsc.subcore_barrier()
            pltpu.sync_copy(acc.at[pl.ds(sub * _RPT, _RPT), :],
                            out_hbm.at[geff, pl.ds(sub * _RPT, _RPT), :])
            plsc.subcore_barrier()


def _scatter_stage(contrib, idx):
    idx2 = idx.reshape(_EPAD // 128, 128)
    zeros = jnp.zeros((_RPT, 8), jnp.float32)
    mesh = plsc.VectorSubcoreMesh(core_axis_name="c", subcore_axis_name="s")
    f = pl.kernel(
        _scatter_body,
        mesh=mesh,
        compiler_params=pltpu.CompilerParams(use_tc_tiling_on_sc=False),
        out_type=jax.ShapeDtypeStruct((_NGRP, _ROWSP, 8), jnp.float32),
        scratch_types=[
            pltpu.VMEM_SHARED((_ROWSP, 8), jnp.float32),
            pltpu.VMEM((10, 128), jnp.int32),
            pltpu.VMEM((_SUP, 8), jnp.float32),
            pltpu.SemaphoreType.DMA,
            pltpu.SemaphoreType.DMA,
            pltpu.SemaphoreType.DMA,
        ],
    )
    return f(contrib, idx2, zeros)


def _edge_stage(positions, numbers, edge_indices):
    src = edge_indices[0]
    dst = edge_indices[1]
    table = jnp.concatenate(
        [positions, numbers.astype(jnp.float32)[:, None],
         jnp.zeros((N, _TW - 4), jnp.float32)], axis=1)
    S, D = _gather_stage(table, src, dst)
    contrib, idx = _edge_math_stage(S, D, dst)
    return _scatter_stage(contrib, idx)


def kernel(positions, cells, numbers, edge_indices, edge_shifts, ptr,
           Wc, bc, Wp, bp, W1, b1, W2, b2, W3, b3):
    del cells, edge_shifts
    numbers = numbers.astype(jnp.int32)
    edge_indices = edge_indices.astype(jnp.int32)
    one_hot = jax.nn.one_hot(numbers, A, dtype=positions.dtype)
    compositions = one_hot.reshape(B, N // B, A).sum(axis=1)
    energies = compositions @ Wc.T + bc

    c = _edge_stage(positions, numbers, edge_indices)
    eatom = _dense_stage(c, Wp, W1, b1, W2, b2, W3)
    per_struct = eatom.reshape(B, N // B).sum(axis=1)
    extra = jnp.float32(N // B) * (bp[0] + b3[0])
    return energies + (per_struct + extra)[:, None]
